# baseline (speedup 1.0000x reference)
import jax
import jax.numpy as jnp
from jax.experimental import pallas as pl
from jax.experimental.pallas import tpu as pltpu

_H = 12
_S = 2048
_D = 768
_FB = 512
_NF = _S // _FB


def _fused_body(p_ref, h_ref, hm_ref, thr_ref, temp_ref, out_ref, acc):
    r = pl.program_id(0)

    @pl.when(r < _H)
    def _():
        acc[pl.ds(r, 1), :] = jnp.sum(p_ref[0, 0], axis=0, keepdims=True)

    @pl.when(r >= _H)
    def _():
        b = r - _H
        hm = hm_ref[...]
        cs = acc[:, pl.ds(b * _FB, _FB)]
        imp = jax.lax.dot_general(
            cs, hm,
            dimension_numbers=(((0,), (0,)), ((), ())),
            preferred_element_type=jnp.float32,
        ) / jnp.sum(hm)
        row = jax.lax.broadcasted_iota(jnp.int32, imp.shape, 0) + b * _FB
        imp = jnp.where(row == 0, imp + 100.0, imp)
        mask = jax.nn.sigmoid((imp - thr_ref[0, 0]) / temp_ref[0, 0])
        out_ref[0] = h_ref[0] * mask


def kernel(hidden_states, attention_probs, head_masks, attention_mask, temp, threshold):
    out = pl.pallas_call(
        _fused_body,
        grid=(_H + _NF,),
        in_specs=[
            pl.BlockSpec((1, 1, _S, _S),
                         lambda r: (0, jnp.minimum(r, _H - 1), 0, 0)),
            pl.BlockSpec((1, _FB, _D),
                         lambda r: (0, jnp.maximum(r - _H, 0), 0)),
            pl.BlockSpec((_H, 1), lambda r: (0, 0)),
            pl.BlockSpec((1, 1), lambda r: (0, 0)),
            pl.BlockSpec((1, 1), lambda r: (0, 0)),
        ],
        out_specs=pl.BlockSpec((1, _FB, _D),
                               lambda r: (0, jnp.maximum(r - _H, 0), 0)),
        out_shape=jax.ShapeDtypeStruct((1, _S, _D), jnp.float32),
        scratch_shapes=[pltpu.VMEM((_H, _S), jnp.float32)],
        compiler_params=pltpu.CompilerParams(
            dimension_semantics=("arbitrary",),
        ),
    )(attention_probs, hidden_states, head_masks.reshape(_H, 1),
      threshold.reshape(1, 1), temp.reshape(1, 1))

    return (out, attention_mask)


# speedup vs baseline: 1.0255x; 1.0255x over previous
import jax
import jax.numpy as jnp
from jax.experimental import pallas as pl
from jax.experimental.pallas import tpu as pltpu

_H = 12
_S = 2048
_D = 768
_FB = 512
_NF = _S // _FB


def _fused_body(p_ref, h_ref, hm_ref, am_ref, thr_ref, temp_ref,
                out_ref, am_out_ref, acc):
    r = pl.program_id(0)

    @pl.when(r < _H)
    def _():
        acc[pl.ds(r, 1), :] = jnp.sum(p_ref[0, 0], axis=0, keepdims=True)

    @pl.when(r == 0)
    def _():
        am_out_ref[...] = am_ref[...]

    @pl.when(r >= _H)
    def _():
        b = r - _H
        hm = hm_ref[...]
        cs = acc[:, pl.ds(b * _FB, _FB)]
        imp = jax.lax.dot_general(
            cs, hm,
            dimension_numbers=(((0,), (1,)), ((), ())),
            preferred_element_type=jnp.float32,
        ) / jnp.sum(hm)
        row = jax.lax.broadcasted_iota(jnp.int32, imp.shape, 0) + b * _FB
        imp = jnp.where(row == 0, imp + 100.0, imp)
        mask = jax.nn.sigmoid((imp - thr_ref[0, 0]) / temp_ref[0, 0])
        out_ref[0] = h_ref[0] * mask


def kernel(hidden_states, attention_probs, head_masks, attention_mask, temp, threshold):
    out, am_out = pl.pallas_call(
        _fused_body,
        grid=(_H + _NF,),
        in_specs=[
            pl.BlockSpec((1, 1, _S, _S),
                         lambda r: (0, jnp.minimum(r, _H - 1), 0, 0)),
            pl.BlockSpec((1, _FB, _D),
                         lambda r: (0, jnp.maximum(r - _H, 0), 0)),
            pl.BlockSpec((1, _H), lambda r: (0, 0)),
            pl.BlockSpec((1, 1, 1, _S), lambda r: (0, 0, 0, 0)),
            pl.BlockSpec((1, 1), lambda r: (0, 0)),
            pl.BlockSpec((1, 1), lambda r: (0, 0)),
        ],
        out_specs=[
            pl.BlockSpec((1, _FB, _D),
                         lambda r: (0, jnp.maximum(r - _H, 0), 0)),
            pl.BlockSpec((1, 1, 1, _S), lambda r: (0, 0, 0, 0)),
        ],
        out_shape=[
            jax.ShapeDtypeStruct((1, _S, _D), jnp.float32),
            jax.ShapeDtypeStruct((1, 1, 1, _S), jnp.float32),
        ],
        scratch_shapes=[pltpu.VMEM((_H, _S), jnp.float32)],
        compiler_params=pltpu.CompilerParams(
            dimension_semantics=("arbitrary",),
        ),
    )(attention_probs, hidden_states, head_masks.reshape(1, _H),
      attention_mask, threshold.reshape(1, 1), temp.reshape(1, 1))

    return (out, am_out)


# speedup vs baseline: 1.0420x; 1.0161x over previous
import jax
import jax.numpy as jnp
from jax.experimental import pallas as pl
from jax.experimental.pallas import tpu as pltpu

_H = 12
_S = 2048
_D = 768
_FB = 2048
_NF = _S // _FB


def _fused_body(p_ref, h_ref, hm_ref, am_ref, thr_ref, temp_ref,
                out_ref, am_out_ref, acc):
    r = pl.program_id(0)

    @pl.when(r < _H)
    def _():
        acc[pl.ds(r, 1), :] = jnp.sum(p_ref[0, 0], axis=0, keepdims=True)

    @pl.when(r == 0)
    def _():
        am_out_ref[...] = am_ref[...]

    @pl.when(r >= _H)
    def _():
        b = r - _H
        hm = hm_ref[...]
        cs = acc[:, pl.ds(b * _FB, _FB)]
        imp = jax.lax.dot_general(
            cs, hm,
            dimension_numbers=(((0,), (1,)), ((), ())),
            preferred_element_type=jnp.float32,
        ) / jnp.sum(hm)
        row = jax.lax.broadcasted_iota(jnp.int32, imp.shape, 0) + b * _FB
        imp = jnp.where(row == 0, imp + 100.0, imp)
        mask = jax.nn.sigmoid((imp - thr_ref[0, 0]) / temp_ref[0, 0])
        out_ref[0] = h_ref[0] * mask


def kernel(hidden_states, attention_probs, head_masks, attention_mask, temp, threshold):
    out, am_out = pl.pallas_call(
        _fused_body,
        grid=(_H + _NF,),
        in_specs=[
            pl.BlockSpec((1, 1, _S, _S),
                         lambda r: (0, jnp.minimum(r, _H - 1), 0, 0)),
            pl.BlockSpec((1, _FB, _D),
                         lambda r: (0, jnp.maximum(r - _H, 0), 0)),
            pl.BlockSpec((1, _H), lambda r: (0, 0)),
            pl.BlockSpec((1, 1, 1, _S), lambda r: (0, 0, 0, 0)),
            pl.BlockSpec((1, 1), lambda r: (0, 0)),
            pl.BlockSpec((1, 1), lambda r: (0, 0)),
        ],
        out_specs=[
            pl.BlockSpec((1, _FB, _D),
                         lambda r: (0, jnp.maximum(r - _H, 0), 0)),
            pl.BlockSpec((1, 1, 1, _S), lambda r: (0, 0, 0, 0)),
        ],
        out_shape=[
            jax.ShapeDtypeStruct((1, _S, _D), jnp.float32),
            jax.ShapeDtypeStruct((1, 1, 1, _S), jnp.float32),
        ],
        scratch_shapes=[pltpu.VMEM((_H, _S), jnp.float32)],
        compiler_params=pltpu.CompilerParams(
            dimension_semantics=("arbitrary",),
        ),
    )(attention_probs, hidden_states, head_masks.reshape(1, _H),
      attention_mask, threshold.reshape(1, 1), temp.reshape(1, 1))

    return (out, am_out)
